# baseline (speedup 1.0000x reference)
import functools

import jax
import jax.numpy as jnp
from jax import lax
from jax.experimental import pallas as pl
from jax.experimental.pallas import tpu as pltpu
from jax.experimental.pallas import tpu_sc as plsc

N_NODES = 10000
D = 128
NC = 2
NS = 16
NW = NC * NS
E = 320000
E_PER_W = E // NW
E_PAD_W = 10240
SEG = 256
NSEG = E_PAD_W // SEG
NPAD = NW * 320
BINW = 320
LKOFF = 8
AGG_R = 336
CAP = 480
CPROC = 464
GCHS = tuple((q * 64, 64) for q in range(7)) + ((448, 16),)


def _bin_body(key_hbm, val_hbm, ov_hbm, ok_hbm, key_s, val_s, cv, ck, cur_s):
    c = lax.axis_index("c")
    s = lax.axis_index("s")
    wid = c * NS + s

    zeros = jnp.zeros((16,), jnp.int32)

    def prefill(i, carry):
        cv[pl.ds(i * 16, 16)] = zeros
        ck[pl.ds(i * 16, 16)] = zeros
        return carry

    lax.fori_loop(0, 32 * CAP // 16, prefill, 0)

    def init_cur(b, carry):
        cur_s[b] = b * CAP
        return carry

    lax.fori_loop(0, 32, init_cur, 0)

    lanes = lax.iota(jnp.int32, 16)

    def seg_body(g, carry):
        base = pl.multiple_of(wid * E_PAD_W + g * SEG, SEG)
        pltpu.sync_copy(key_hbm.at[pl.ds(base, SEG)], key_s)
        pltpu.sync_copy(val_hbm.at[pl.ds(base, SEG)], val_s)
        for v in range(SEG // 16):
            kv = key_s[pl.ds(v * 16, 16)]
            vv = val_s[pl.ds(v * 16, 16)]
            bins = lax.shift_right_logical(kv * 6554, 21)
            lk = kv - bins * BINW + LKOFF
            for lane in range(16):
                b = bins[lane]
                cur = cur_s[b]
                plsc.addupdate(cv.at[pl.ds(cur, 16)],
                               jnp.where(lanes == 0, vv[lane], 0))
                plsc.addupdate(ck.at[pl.ds(cur, 16)],
                               jnp.where(lanes == 0, lk[lane], 0))
                cur_s[b] = cur + 1
        return carry

    lax.fori_loop(0, NSEG, seg_body, 0)

    out_sl = pl.ds(wid * 32 * CAP, 32 * CAP)
    pltpu.sync_copy(cv, ov_hbm.at[out_sl])
    pltpu.sync_copy(ck, ok_hbm.at[out_sl])


def _cdeg_body(ck_hbm, deg_hbm, lk_v, deg_v):
    c = lax.axis_index("c")
    s = lax.axis_index("s")
    t = c * NS + s

    def zero(i, carry):
        deg_v[i, pl.ds(0, 16)] = jnp.zeros((16,), jnp.float32)
        return carry

    lax.fori_loop(0, AGG_R, zero, 0)

    ones = jnp.ones((16,), jnp.float32)

    def cell(w, carry):
        base = pl.multiple_of((w * 32 + t) * CAP, 8)
        pltpu.sync_copy(ck_hbm.at[pl.ds(base, CAP)], lk_v)

        def vreg(v, carry2):
            vec = lk_v[pl.ds(v * 16, 16)]
            for lane in range(16):
                lk = vec[lane]
                plsc.addupdate(deg_v.at[lk, pl.ds(0, 16)], ones)
            return carry2

        lax.fori_loop(0, CPROC // 16, vreg, 0)
        return carry

    lax.fori_loop(0, NW, cell, 0)

    pltpu.sync_copy(deg_v.at[pl.ds(LKOFF, BINW)],
                    deg_hbm.at[pl.ds(t * BINW, BINW)])


def _gath_body(h_hbm, cv_hbm, m_hbm, sv0, sv1, rows, sems):
    c = lax.axis_index("c")
    s = lax.axis_index("s")
    t = c * NS + s
    svs = (sv0, sv1)

    pend = None
    for w in range(NW + 1):
        if w < NW:
            b = w % 2
            pltpu.sync_copy(
                cv_hbm.at[pl.ds(pl.multiple_of((w * 32 + t) * CAP, 8), CAP)],
                svs[b])
            descs = [
                pltpu.async_copy(
                    h_hbm.at[svs[b].at[pl.ds(off, ln)]],
                    rows.at[b].at[pl.ds(off, ln)], sems[b * 8 + q])
                for q, (off, ln) in enumerate(GCHS)
            ]
        if pend is not None:
            for d in pend:
                d.wait()
            pb = (w - 1) % 2
            pltpu.sync_copy(rows.at[pb],
                            m_hbm.at[t].at[pl.ds((w - 1) * CPROC, CPROC)])
        pend = descs if w < NW else None


@functools.cache
def _sc_kernels():
    mesh = plsc.VectorSubcoreMesh(core_axis_name="c", subcore_axis_name="s")
    i32 = jnp.int32
    bink = pl.kernel(
        _bin_body,
        out_type=(
            jax.ShapeDtypeStruct((NW * 32 * CAP,), i32),
            jax.ShapeDtypeStruct((NW * 32 * CAP,), i32),
        ),
        mesh=mesh,
        scratch_types=[
            pltpu.VMEM((SEG,), i32),
            pltpu.VMEM((SEG,), i32),
            pltpu.VMEM((32 * CAP,), i32),
            pltpu.VMEM((32 * CAP,), i32),
            pltpu.SMEM((32,), i32),
        ],
    )
    cdeg = pl.kernel(
        _cdeg_body,
        out_type=jax.ShapeDtypeStruct((NPAD, 16), jnp.float32),
        mesh=mesh,
        scratch_types=[
            pltpu.VMEM((CAP,), i32),
            pltpu.VMEM((AGG_R, 16), jnp.float32),
        ],
    )
    gath = pl.kernel(
        _gath_body,
        out_type=jax.ShapeDtypeStruct((NW, NW * CPROC, D), jnp.float32),
        mesh=mesh,
        scratch_types=[
            pltpu.VMEM((CAP,), i32),
            pltpu.VMEM((CAP,), i32),
            pltpu.VMEM((2, CPROC, D), jnp.float32),
            [pltpu.SemaphoreType.DMA] * (2 * len(GCHS)),
        ],
    )
    return bink, cdeg, gath


def _bin_sc(key, val):
    return _sc_kernels()[0](key, val)


def _cdeg_sc(ck):
    return _sc_kernels()[1](ck)


def _gath_sc(h, cv):
    return _sc_kernels()[2](h, cv)


ECH = 928
NCH = NW * CPROC // ECH


def _spmm_tc(lk_ref, m_ref, o_ref, acc):
    ch = pl.program_id(1)

    @pl.when(ch == 0)
    def _():
        acc[...] = jnp.zeros_like(acc)

    rows = lax.broadcasted_iota(jnp.int32, (AGG_R, ECH), 0)
    oh = jnp.where(rows == lk_ref[0], 1.0, 0.0)
    acc[...] += jnp.dot(oh, m_ref[0], preferred_element_type=jnp.float32)

    @pl.when(ch == NCH - 1)
    def _():
        o_ref[0] = acc[LKOFF:LKOFF + BINW, :]


def _spmm(lk_r, m):
    out = pl.pallas_call(
        _spmm_tc,
        grid=(NW, NCH),
        in_specs=[
            pl.BlockSpec((1, 1, ECH), lambda b, c: (b * NCH + c, 0, 0)),
            pl.BlockSpec((1, ECH, D), lambda b, c: (b, c, 0)),
        ],
        out_specs=pl.BlockSpec((1, BINW, D), lambda b, c: (b, 0, 0)),
        out_shape=jax.ShapeDtypeStruct((NW, BINW, D), jnp.float32),
        scratch_shapes=[pltpu.VMEM((AGG_R, D), jnp.float32)],
    )(lk_r, m)
    return out.reshape(NPAD, D)


def _prep_tc(x_ref, dego_ref, g_ref):
    out_s = lax.rsqrt(jnp.maximum(dego_ref[:, 0:1], 1.0))
    g_ref[...] = (x_ref[...] * out_s).astype(g_ref.dtype)


def _layer_tc(agg_ref, degi_ref, dego_ref, w_ref, b_ref, o_ref, *,
              relu, scale_out):
    in_s = lax.rsqrt(jnp.maximum(degi_ref[:, 0:1], 1.0))
    z = jnp.dot(agg_ref[...] * in_s, w_ref[...],
                preferred_element_type=jnp.float32) + b_ref[...]
    if relu:
        z = jnp.maximum(z, 0.0)
    if scale_out:
        z = z * lax.rsqrt(jnp.maximum(dego_ref[:, 0:1], 1.0))
    o_ref[...] = z.astype(o_ref.dtype)


def _tc_call(fn, out_shape, *args, dtype=jnp.float32):
    return pl.pallas_call(
        fn, out_shape=jax.ShapeDtypeStruct(out_shape, dtype))(*args)


def kernel(x, edge_index, W1, b1, W2, b2, W3, b3):
    pad = ((0, 0), (0, E_PAD_W - E_PER_W))
    src = jnp.pad(edge_index[0].reshape(NW, E_PER_W), pad,
                  constant_values=N_NODES).reshape(-1)
    dst = jnp.pad(edge_index[1].reshape(NW, E_PER_W), pad,
                  constant_values=N_NODES).reshape(-1)
    x_pad = jnp.pad(x, ((0, NPAD - N_NODES), (0, 0)))
    b1r = b1.reshape(1, D)
    b2r = b2.reshape(1, D)
    b3r = b3.reshape(1, 64)

    cvA, ckA = _bin_sc(dst, src)
    _, ckB = _bin_sc(src, dst)
    deg_in = _cdeg_sc(ckA)
    deg_out = _cdeg_sc(ckB)

    lk_r = (ckA.reshape(NW, NW, CAP)[:, :, :CPROC]
            .transpose(1, 0, 2).reshape(NW * NCH, 1, ECH))

    g = _tc_call(_prep_tc, (NPAD, D), x_pad, deg_out)
    p = _spmm(lk_r, _gath_sc(g, cvA))
    g = _tc_call(functools.partial(_layer_tc, relu=True, scale_out=True),
                 (NPAD, D), p, deg_in, deg_out, W1, b1r)
    p = _spmm(lk_r, _gath_sc(g, cvA))
    g = _tc_call(functools.partial(_layer_tc, relu=True, scale_out=True),
                 (NPAD, D), p, deg_in, deg_out, W2, b2r)
    p = _spmm(lk_r, _gath_sc(g, cvA))
    out = _tc_call(functools.partial(_layer_tc, relu=False, scale_out=False),
                   (NPAD, 64), p, deg_in, deg_out, W3, b3r)
    return out[:N_NODES]


# speedup vs baseline: 1.1328x; 1.1328x over previous
import functools

import jax
import jax.numpy as jnp
from jax import lax
from jax.experimental import pallas as pl
from jax.experimental.pallas import tpu as pltpu
from jax.experimental.pallas import tpu_sc as plsc

N_NODES = 10000
D = 128
NC = 2
NS = 16
NW = NC * NS
E = 320000
E_PER_W = E // NW
E_PAD_W = 10240
SEG = 256
NSEG = E_PAD_W // SEG
NPAD = NW * 320
BINW = 320
LKOFF = 8
AGG_R = 336
CAP = 480
CPROC = 464
GCHS = tuple((q * 64, 64) for q in range(7)) + ((448, 16),)


def _bin_body(key_hbm, val_hbm, ov_hbm, ok_hbm, key_s, val_s, cv, ck, cur_s):
    c = lax.axis_index("c")
    s = lax.axis_index("s")
    wid = c * NS + s

    zeros = jnp.zeros((16,), jnp.int32)

    def prefill(i, carry):
        cv[pl.ds(i * 16, 16)] = zeros
        ck[pl.ds(i * 16, 16)] = zeros
        return carry

    lax.fori_loop(0, 32 * CAP // 16, prefill, 0)

    def init_cur(b, carry):
        cur_s[b] = b * CAP
        return carry

    lax.fori_loop(0, 32, init_cur, 0)

    lanes = lax.iota(jnp.int32, 16)

    def seg_body(g, carry):
        base = pl.multiple_of(wid * E_PAD_W + g * SEG, SEG)
        pltpu.sync_copy(key_hbm.at[pl.ds(base, SEG)], key_s)
        pltpu.sync_copy(val_hbm.at[pl.ds(base, SEG)], val_s)
        for v in range(SEG // 16):
            kv = key_s[pl.ds(v * 16, 16)]
            vv = val_s[pl.ds(v * 16, 16)]
            bins = lax.shift_right_logical(kv * 6554, 21)
            lk = kv - bins * BINW + LKOFF
            for lane in range(16):
                b = bins[lane]
                cur = cur_s[b]
                plsc.addupdate(cv.at[pl.ds(cur, 16)],
                               jnp.where(lanes == 0, vv[lane], 0))
                plsc.addupdate(ck.at[pl.ds(cur, 16)],
                               jnp.where(lanes == 0, lk[lane], 0))
                cur_s[b] = cur + 1
        return carry

    lax.fori_loop(0, NSEG, seg_body, 0)

    out_sl = pl.ds(wid * 32 * CAP, 32 * CAP)
    pltpu.sync_copy(cv, ov_hbm.at[out_sl])
    pltpu.sync_copy(ck, ok_hbm.at[out_sl])


def _cdeg_body(ck_hbm, deg_hbm, lk_v, deg_v):
    c = lax.axis_index("c")
    s = lax.axis_index("s")
    t = c * NS + s

    def zero(i, carry):
        deg_v[i, pl.ds(0, 16)] = jnp.zeros((16,), jnp.float32)
        return carry

    lax.fori_loop(0, AGG_R, zero, 0)

    ones = jnp.ones((16,), jnp.float32)

    def cell(w, carry):
        base = pl.multiple_of((w * 32 + t) * CAP, 8)
        pltpu.sync_copy(ck_hbm.at[pl.ds(base, CAP)], lk_v)

        def vreg(v, carry2):
            vec = lk_v[pl.ds(v * 16, 16)]
            for lane in range(16):
                lk = vec[lane]
                plsc.addupdate(deg_v.at[lk, pl.ds(0, 16)], ones)
            return carry2

        lax.fori_loop(0, CPROC // 16, vreg, 0)
        return carry

    lax.fori_loop(0, NW, cell, 0)

    pltpu.sync_copy(deg_v.at[pl.ds(LKOFF, BINW)],
                    deg_hbm.at[pl.ds(t * BINW, BINW)])


def _agg_body(h_hbm, cv_hbm, ck_hbm, agg_hbm, sv, lk_v, rows, agg_v, sems):
    c = lax.axis_index("c")
    s = lax.axis_index("s")
    t = c * NS + s

    def zero(i, carry):
        for c8 in range(8):
            agg_v[i, pl.ds(c8 * 16, 16)] = jnp.zeros((16,), jnp.float32)
        return carry

    lax.fori_loop(0, AGG_R, zero, 0)

    def cell(w, carry):
        base = pl.multiple_of((w * 32 + t) * CAP, 8)
        pltpu.sync_copy(cv_hbm.at[pl.ds(base, CAP)], sv)
        pltpu.sync_copy(ck_hbm.at[pl.ds(base, CAP)], lk_v)
        descs = [
            pltpu.async_copy(
                h_hbm.at[sv.at[pl.ds(off, ln)]],
                rows.at[pl.ds(off, ln)], sems[q])
            for q, (off, ln) in enumerate(GCHS)
        ]
        for d in descs:
            d.wait()

        def vreg(v, carry2):
            vec = lk_v[pl.ds(v * 16, 16)]
            for lane in range(16):
                lk = vec[lane]
                e = v * 16 + lane
                for c8 in range(8):
                    plsc.addupdate(agg_v.at[lk, pl.ds(c8 * 16, 16)],
                                   rows[e, pl.ds(c8 * 16, 16)])
            return carry2

        lax.fori_loop(0, CPROC // 16, vreg, 0)
        return carry

    lax.fori_loop(0, NW, cell, 0)

    pltpu.sync_copy(agg_v.at[pl.ds(LKOFF, BINW)],
                    agg_hbm.at[pl.ds(t * BINW, BINW)])


@functools.cache
def _sc_kernels():
    mesh = plsc.VectorSubcoreMesh(core_axis_name="c", subcore_axis_name="s")
    i32 = jnp.int32
    bink = pl.kernel(
        _bin_body,
        out_type=(
            jax.ShapeDtypeStruct((NW * 32 * CAP,), i32),
            jax.ShapeDtypeStruct((NW * 32 * CAP,), i32),
        ),
        mesh=mesh,
        scratch_types=[
            pltpu.VMEM((SEG,), i32),
            pltpu.VMEM((SEG,), i32),
            pltpu.VMEM((32 * CAP,), i32),
            pltpu.VMEM((32 * CAP,), i32),
            pltpu.SMEM((32,), i32),
        ],
    )
    cdeg = pl.kernel(
        _cdeg_body,
        out_type=jax.ShapeDtypeStruct((NPAD, 16), jnp.float32),
        mesh=mesh,
        scratch_types=[
            pltpu.VMEM((CAP,), i32),
            pltpu.VMEM((AGG_R, 16), jnp.float32),
        ],
    )
    agg = pl.kernel(
        _agg_body,
        out_type=jax.ShapeDtypeStruct((NPAD, D), jnp.float32),
        mesh=mesh,
        scratch_types=[
            pltpu.VMEM((CAP,), i32),
            pltpu.VMEM((CAP,), i32),
            pltpu.VMEM((CPROC, D), jnp.float32),
            pltpu.VMEM((AGG_R, D), jnp.float32),
            [pltpu.SemaphoreType.DMA] * len(GCHS),
        ],
    )
    return bink, cdeg, agg


def _bin_sc(key, val):
    return _sc_kernels()[0](key, val)


def _cdeg_sc(ck):
    return _sc_kernels()[1](ck)


def _agg_sc(h, cv, ck):
    return _sc_kernels()[2](h, cv, ck)


ECH = 928
NCH = NW * CPROC // ECH


def _spmm_tc(lk_ref, m_ref, o_ref, acc):
    ch = pl.program_id(1)

    @pl.when(ch == 0)
    def _():
        acc[...] = jnp.zeros_like(acc)

    rows = lax.broadcasted_iota(jnp.int32, (AGG_R, ECH), 0)
    oh = jnp.where(rows == lk_ref[0], 1.0, 0.0)
    acc[...] += jnp.dot(oh, m_ref[0], preferred_element_type=jnp.float32)

    @pl.when(ch == NCH - 1)
    def _():
        o_ref[0] = acc[LKOFF:LKOFF + BINW, :]


def _spmm(lk_r, m):
    out = pl.pallas_call(
        _spmm_tc,
        grid=(NW, NCH),
        in_specs=[
            pl.BlockSpec((1, 1, ECH), lambda b, c: (b * NCH + c, 0, 0)),
            pl.BlockSpec((1, ECH, D), lambda b, c: (b, c, 0)),
        ],
        out_specs=pl.BlockSpec((1, BINW, D), lambda b, c: (b, 0, 0)),
        out_shape=jax.ShapeDtypeStruct((NW, BINW, D), jnp.float32),
        scratch_shapes=[pltpu.VMEM((AGG_R, D), jnp.float32)],
    )(lk_r, m)
    return out.reshape(NPAD, D)


def _prep_tc(x_ref, dego_ref, g_ref):
    out_s = lax.rsqrt(jnp.maximum(dego_ref[:, 0:1], 1.0))
    g_ref[...] = (x_ref[...] * out_s).astype(g_ref.dtype)


def _layer_tc(agg_ref, degi_ref, dego_ref, w_ref, b_ref, o_ref, *,
              relu, scale_out):
    in_s = lax.rsqrt(jnp.maximum(degi_ref[:, 0:1], 1.0))
    z = jnp.dot(agg_ref[...] * in_s, w_ref[...],
                preferred_element_type=jnp.float32) + b_ref[...]
    if relu:
        z = jnp.maximum(z, 0.0)
    if scale_out:
        z = z * lax.rsqrt(jnp.maximum(dego_ref[:, 0:1], 1.0))
    o_ref[...] = z.astype(o_ref.dtype)


def _tc_call(fn, out_shape, *args, dtype=jnp.float32):
    return pl.pallas_call(
        fn, out_shape=jax.ShapeDtypeStruct(out_shape, dtype))(*args)


def kernel(x, edge_index, W1, b1, W2, b2, W3, b3):
    pad = ((0, 0), (0, E_PAD_W - E_PER_W))
    src = jnp.pad(edge_index[0].reshape(NW, E_PER_W), pad,
                  constant_values=N_NODES).reshape(-1)
    dst = jnp.pad(edge_index[1].reshape(NW, E_PER_W), pad,
                  constant_values=N_NODES).reshape(-1)
    x_pad = jnp.pad(x, ((0, NPAD - N_NODES), (0, 0)))
    b1r = b1.reshape(1, D)
    b2r = b2.reshape(1, D)
    b3r = b3.reshape(1, 64)

    cvA, ckA = _bin_sc(dst, src)
    _, ckB = _bin_sc(src, dst)
    deg_in = _cdeg_sc(ckA)
    deg_out = _cdeg_sc(ckB)

    g = _tc_call(_prep_tc, (NPAD, D), x_pad, deg_out)
    p = _agg_sc(g, cvA, ckA)
    g = _tc_call(functools.partial(_layer_tc, relu=True, scale_out=True),
                 (NPAD, D), p, deg_in, deg_out, W1, b1r)
    p = _agg_sc(g, cvA, ckA)
    g = _tc_call(functools.partial(_layer_tc, relu=True, scale_out=True),
                 (NPAD, D), p, deg_in, deg_out, W2, b2r)
    p = _agg_sc(g, cvA, ckA)
    out = _tc_call(functools.partial(_layer_tc, relu=False, scale_out=False),
                   (NPAD, 64), p, deg_in, deg_out, W3, b3r)
    return out[:N_NODES]
